# baseline (speedup 1.0000x reference)
import functools

import jax
import jax.numpy as jnp
from jax import lax
from jax.experimental import pallas as pl
from jax.experimental.pallas import tpu as pltpu
from jax.experimental.pallas import tpu_sc as plsc

HIDDEN = 128
LANES = 16
CHUNK = 128


@functools.cache
def _build(n_tokens):
    info = plsc.get_sparse_core_info()
    nw = info.num_cores * info.num_subcores
    per_w = n_tokens // nw
    n_chunks = per_w // CHUNK
    chunks_row = CHUNK // HIDDEN
    mesh = plsc.VectorSubcoreMesh(core_axis_name="c", subcore_axis_name="s")

    rows_f32 = pltpu.VMEM((CHUNK, HIDDEN), jnp.float32)

    @functools.partial(
        pl.kernel,
        out_type=jax.ShapeDtypeStruct((n_tokens, HIDDEN), jnp.float32),
        mesh=mesh,
        scratch_types=[
            pltpu.VMEM((n_chunks, CHUNK), jnp.int32),
            pltpu.VMEM((n_chunks, CHUNK), jnp.int32),
            pltpu.VMEM((n_chunks, CHUNK), jnp.int32),
            rows_f32, rows_f32,
            rows_f32, rows_f32,
            rows_f32, rows_f32,
            pltpu.VMEM((2, HIDDEN), jnp.float32),
            pltpu.SemaphoreType.DMA,
            pltpu.SemaphoreType.DMA,
            pltpu.SemaphoreType.DMA,
            pltpu.SemaphoreType.DMA,
            pltpu.SemaphoreType.DMA,
            pltpu.SemaphoreType.DMA,
            pltpu.SemaphoreType.DMA,
            pltpu.SemaphoreType.DMA,
            pltpu.SemaphoreType.DMA,
        ],
    )
    def emb_kernel(ids_hbm, pos_hbm, tt_hbm, word_hbm, ptab_hbm, ttab_hbm,
                   out_hbm, idx_w, idx_p, idx_t,
                   rw0, rp0, rw1, rp1, rw2, rp2, ttab_v,
                   gs0, gs1, gs2, ps0, ps1, ps2, os0, os1, os2):
        wid = lax.axis_index("s") * info.num_cores + lax.axis_index("c")
        base = wid * per_w
        idx_row0 = wid * (per_w // HIDDEN)

        pltpu.sync_copy(ttab_hbm, ttab_v)


        nrows = n_chunks * chunks_row
        pltpu.sync_copy(ids_hbm.at[pl.ds(idx_row0, nrows)], idx_w)
        pltpu.sync_copy(pos_hbm.at[pl.ds(idx_row0, nrows)], idx_p)
        pltpu.sync_copy(tt_hbm.at[pl.ds(idx_row0, nrows)], idx_t)

        nbuf = 3
        bufs = [(rw0, rp0), (rw1, rp1), (rw2, rp2)]
        gsems = [gs0, gs1, gs2]
        psems = [ps0, ps1, ps2]
        osems = [os0, os1, os2]
        gcop = [None] * nbuf
        ocop = [None] * nbuf

        def issue(ci):
            b = ci % nbuf
            rw, rp = bufs[b]
            gcop[b] = (
                pltpu.async_copy(word_hbm.at[idx_w.at[ci]], rw, gsems[b]),
                pltpu.async_copy(ptab_hbm.at[idx_p.at[ci]], rp, psems[b]),
            )

        issue(0)
        if n_chunks > 1:
            issue(1)
        for ci in range(n_chunks):
            b = ci % nbuf
            if ci + 2 < n_chunks:
                nb = (ci + 2) % nbuf
                if ocop[nb] is not None:
                    ocop[nb].wait()
                    ocop[nb] = None
                issue(ci + 2)
            with jax.named_scope("gwait"):
                for c in gcop[b]:
                    c.wait()
            rw, rp = bufs[b]

            with jax.named_scope("addloop"):
                @plsc.parallel_loop(0, CHUNK // LANES, 1, unroll=2)
                def add_body(g, rw=rw, rp=rp, ci=ci):
                    ttv = idx_t[ci, pl.ds(g * LANES, LANES)]
                    for l in range(LANES):
                        tt = ttv[l]
                        t = g * LANES + l
                        for j in range(HIDDEN // LANES):
                            sl = pl.ds(j * LANES, LANES)
                            rw[t, sl] = rw[t, sl] + rp[t, sl] + ttab_v[tt, sl]

            ocop[b] = pltpu.async_copy(
                rw, out_hbm.at[pl.ds(base + ci * CHUNK, CHUNK)], osems[b])
        for b in range(nbuf):
            if ocop[b] is not None:
                ocop[b].wait()

    return emb_kernel


@jax.jit
def kernel(input_ids, position_ids, token_type_ids, word_table, pos_table,
           type_table):
    b, s = input_ids.shape
    n = b * s
    ids = input_ids.reshape(n // HIDDEN, HIDDEN).astype(jnp.int32)
    pos = position_ids.reshape(n // HIDDEN, HIDDEN).astype(jnp.int32)
    tts = token_type_ids.reshape(n // HIDDEN, HIDDEN).astype(jnp.int32)
    out = _build(n)(ids, pos, tts, word_table, pos_table, type_table)
    return out.reshape(b, s, HIDDEN)


# speedup vs baseline: 1.1372x; 1.1372x over previous
import functools

import jax
import jax.numpy as jnp
from jax import lax
from jax.experimental import pallas as pl
from jax.experimental.pallas import tpu as pltpu
from jax.experimental.pallas import tpu_sc as plsc

HIDDEN = 128
LANES = 16
CHUNK = 128


@functools.cache
def _build(n_tokens):
    info = plsc.get_sparse_core_info()
    nw = info.num_cores * info.num_subcores
    per_w = n_tokens // nw
    n_chunks = per_w // CHUNK
    chunks_row = CHUNK // HIDDEN
    mesh = plsc.VectorSubcoreMesh(core_axis_name="c", subcore_axis_name="s")

    rows_f32 = pltpu.VMEM((CHUNK, HIDDEN), jnp.float32)

    @functools.partial(
        pl.kernel,
        out_type=jax.ShapeDtypeStruct((n_tokens, HIDDEN), jnp.float32),
        mesh=mesh,
        scratch_types=[
            pltpu.VMEM((n_chunks, CHUNK), jnp.int32),
            pltpu.VMEM((n_chunks, CHUNK), jnp.int32),
            pltpu.VMEM((n_chunks, CHUNK), jnp.int32),
            rows_f32, rows_f32,
            rows_f32, rows_f32,
            rows_f32, rows_f32,
            pltpu.VMEM((2, HIDDEN), jnp.float32),
            pltpu.SemaphoreType.DMA,
            pltpu.SemaphoreType.DMA,
            pltpu.SemaphoreType.DMA,
            pltpu.SemaphoreType.DMA,
            pltpu.SemaphoreType.DMA,
            pltpu.SemaphoreType.DMA,
            pltpu.SemaphoreType.DMA,
            pltpu.SemaphoreType.DMA,
            pltpu.SemaphoreType.DMA,
        ],
    )
    def emb_kernel(ids_hbm, pos_hbm, tt_hbm, word_hbm, ptab_hbm, ttab_hbm,
                   out_hbm, idx_w, idx_p, idx_t,
                   rw0, rp0, rw1, rp1, rw2, rp2, ttab_v,
                   gs0, gs1, gs2, ps0, ps1, ps2, os0, os1, os2):
        wid = lax.axis_index("s") * info.num_cores + lax.axis_index("c")
        base = wid * per_w
        idx_row0 = wid * (per_w // HIDDEN)

        pltpu.sync_copy(ttab_hbm, ttab_v)


        nrows = n_chunks * chunks_row
        pltpu.sync_copy(ids_hbm.at[pl.ds(idx_row0, nrows)], idx_w)
        pltpu.sync_copy(pos_hbm.at[pl.ds(idx_row0, nrows)], idx_p)
        pltpu.sync_copy(tt_hbm.at[pl.ds(idx_row0, nrows)], idx_t)

        nbuf = 3
        bufs = [(rw0, rp0), (rw1, rp1), (rw2, rp2)]
        gsems = [gs0, gs1, gs2]
        psems = [ps0, ps1, ps2]
        osems = [os0, os1, os2]
        gcop = [None] * nbuf
        ocop = [None] * nbuf

        def issue(ci):
            b = ci % nbuf
            rw, rp = bufs[b]
            gcop[b] = (
                pltpu.async_copy(word_hbm.at[idx_w.at[ci]], rw, gsems[b]),
                pltpu.async_copy(ptab_hbm.at[idx_p.at[ci]], rp, psems[b]),
            )

        issue(0)
        if n_chunks > 1:
            issue(1)
        for ci in range(n_chunks):
            b = ci % nbuf
            if ci + 2 < n_chunks:
                nb = (ci + 2) % nbuf
                if ocop[nb] is not None:
                    ocop[nb].wait()
                    ocop[nb] = None
                issue(ci + 2)
            with jax.named_scope("gwait"):
                for c in gcop[b]:
                    c.wait()
            rw, rp = bufs[b]

            def add_body(g, carry, rw=rw, rp=rp, ci=ci):
                ttv = idx_t[ci, pl.ds(g * LANES, LANES)]
                for l in range(LANES):
                    tt = ttv[l]
                    t = g * LANES + l
                    for j in range(HIDDEN // LANES):
                        sl = pl.ds(j * LANES, LANES)
                        rw[t, sl] = rw[t, sl] + rp[t, sl] + ttab_v[tt, sl]
                return carry

            with jax.named_scope("addloop"):
                lax.fori_loop(0, CHUNK // LANES, add_body, 0)
            ocop[b] = pltpu.async_copy(
                rw, out_hbm.at[pl.ds(base + ci * CHUNK, CHUNK)], osems[b])
        for b in range(nbuf):
            if ocop[b] is not None:
                ocop[b].wait()

    return emb_kernel


@jax.jit
def kernel(input_ids, position_ids, token_type_ids, word_table, pos_table,
           type_table):
    b, s = input_ids.shape
    n = b * s
    ids = input_ids.reshape(n // HIDDEN, HIDDEN).astype(jnp.int32)
    pos = position_ids.reshape(n // HIDDEN, HIDDEN).astype(jnp.int32)
    tts = token_type_ids.reshape(n // HIDDEN, HIDDEN).astype(jnp.int32)
    out = _build(n)(ids, pos, tts, word_table, pos_table, type_table)
    return out.reshape(b, s, HIDDEN)
